# Initial kernel scaffold; baseline (speedup 1.0000x reference)
import jax
import jax.numpy as jnp
from jax.experimental import pallas as pl


def kernel(x_s, x_d, edge_index, W_rel, b_rel, W_root):
    raise NotImplementedError("write your pallas kernel here")


# speedup vs baseline: 159.2749x; 159.2749x over previous
import functools

import jax
import jax.numpy as jnp
from jax import lax
from jax.experimental import pallas as pl
from jax.experimental.pallas import tpu as pltpu
from jax.experimental.pallas import tpu_sc as plsc

_N_SRC = 100000
_N_DST = 100000
_N_EDGE = 6400000
_ROW = 128
_R = _N_EDGE // _ROW
_K = 16
_UNITS = _R // _K
_NCORES = 2
_NSUB = 16
_NW = _NCORES * _NSUB
_UBASE = _UNITS // _NW
_UREM = _UNITS % _NW
_PAD = 100096
_ZS = _PAD // _NSUB


def _sc_aggregate(xs_flat, src2, dst2):
  mesh = plsc.VectorSubcoreMesh(core_axis_name="c", subcore_axis_name="s")

  @functools.partial(
      pl.kernel,
      out_type=jax.ShapeDtypeStruct((_NCORES * _PAD,), jnp.float32),
      mesh=mesh,
      compiler_params=pltpu.CompilerParams(needs_layout_passes=False),
      scratch_types=[
          pltpu.VMEM((_N_SRC,), jnp.float32),
          pltpu.VMEM((_K, _ROW), jnp.int32),
          pltpu.VMEM((_K, _ROW), jnp.int32),
          pltpu.VMEM((_K, _ROW), jnp.float32),
          pltpu.VMEM((2048,), jnp.float32),
          pltpu.VMEM_SHARED((_PAD,), jnp.float32),
      ],
  )
  def body(xs_hbm, src_hbm, dst_hbm, out_hbm,
           xs_v, src_v, dst_v, val_v, zb_v, agg_sh):
    c = lax.axis_index("c")
    s = lax.axis_index("s")
    wid = s * _NCORES + c

    zv = jnp.zeros((16,), jnp.float32)
    for i in range(2048 // 16):
      zb_v[pl.ds(i * 16, 16)] = zv
    for j in range(_ZS // 2048):
      pltpu.sync_copy(zb_v, agg_sh.at[pl.ds(s * _ZS + j * 2048, 2048)])
    pltpu.sync_copy(zb_v.at[pl.ds(0, _ZS % 2048)],
                    agg_sh.at[pl.ds(s * _ZS + (_ZS // 2048) * 2048,
                                    _ZS % 2048)])
    pltpu.sync_copy(xs_hbm, xs_v)
    plsc.subcore_barrier()

    nunits = _UBASE + jnp.where(wid < _UREM, 1, 0)
    start = wid * _UBASE + jnp.minimum(wid, _UREM)

    def unit(u, carry):
      row0 = (start + u) * _K
      pltpu.sync_copy(src_hbm.at[pl.ds(row0, _K)], src_v)
      pltpu.sync_copy(dst_hbm.at[pl.ds(row0, _K)], dst_v)
      for k in range(_K):
        for i in range(_ROW // 16):
          idx = src_v[k, pl.ds(i * 16, 16)]
          val_v[k, pl.ds(i * 16, 16)] = plsc.load_gather(xs_v, [idx])
      for k in range(_K):
        pltpu.sync_copy(val_v.at[k], agg_sh.at[dst_v.at[k]], add=True)
      return carry

    lax.fori_loop(0, nunits, unit, 0)

    plsc.subcore_barrier()
    for j in range(_ZS // 2048):
      pltpu.sync_copy(agg_sh.at[pl.ds(s * _ZS + j * 2048, 2048)], zb_v)
      pltpu.sync_copy(zb_v, out_hbm.at[pl.ds(c * _PAD + s * _ZS + j * 2048,
                                             2048)])
    tail = _ZS % 2048
    toff = s * _ZS + (_ZS // 2048) * 2048
    pltpu.sync_copy(agg_sh.at[pl.ds(toff, tail)], zb_v.at[pl.ds(0, tail)])
    pltpu.sync_copy(zb_v.at[pl.ds(0, tail)],
                    out_hbm.at[pl.ds(c * _PAD + toff, tail)])

  return body(xs_flat, src2, dst2)


def _finalize(agg3, xd_pad, w_rel, b_rel, w_root):
  def body(a_ref, xd_ref, wr_ref, br_ref, wk_ref, o_ref):
    o_ref[...] = ((a_ref[0] + a_ref[1]) * wr_ref[0, 0]
                  + br_ref[0, 0] + xd_ref[...] * wk_ref[0, 0])

  scalar = pl.BlockSpec(memory_space=pltpu.SMEM)
  return pl.pallas_call(
      body,
      out_shape=jax.ShapeDtypeStruct(xd_pad.shape, jnp.float32),
      in_specs=[pl.BlockSpec(memory_space=pltpu.VMEM),
                pl.BlockSpec(memory_space=pltpu.VMEM),
                scalar, scalar, scalar],
  )(agg3, xd_pad, w_rel, b_rel, w_root)


def kernel(x_s, x_d, edge_index, W_rel, b_rel, W_root):
  src2 = edge_index[0].astype(jnp.int32).reshape(_R, _ROW)
  dst2 = edge_index[1].astype(jnp.int32).reshape(_R, _ROW)
  xs_flat = x_s.reshape(_N_SRC)

  agg = _sc_aggregate(xs_flat, src2, dst2)
  agg3 = agg.reshape(_NCORES, _PAD // 128, 128)
  xd_pad = jnp.pad(x_d.reshape(_N_DST), (0, _PAD - _N_DST)).reshape(
      _PAD // 128, 128)
  y = _finalize(agg3, xd_pad, W_rel.reshape(1, 1), b_rel.reshape(1, 1),
                W_root.reshape(1, 1))
  return y.reshape(_PAD)[:_N_DST].reshape(_N_DST, 1)
